# baseline (speedup 1.0000x reference)
import jax
import jax.numpy as jnp
from jax import lax
from jax.experimental import pallas as pl
from jax.experimental.pallas import tpu as pltpu, tpu_sc as plsc

N_NODES = 10000
N_EDGES = 320000
D_FEAT = 128
HEADS1 = 8
CH1 = 32
NUM_CLASSES = 40

NSC = 2
NT = 16
LANES = 16

C1 = 128
C2 = 128
EP = 327680
EPT1 = EP // NT
NCH1 = EPT1 // C1
EPT2 = EP // (NSC * NT)
NCH2 = EPT2 // C2

NPAD = 10112
RPT = NPAD // NT
ACC1W = 144
ACC2W = 64

_f32 = jnp.float32
_i32 = jnp.int32


def _tc1_body(x_ref, w1_ref, ac_ref, mt_ref, atd_ref):
    h = jnp.dot(x_ref[...], w1_ref[...], preferred_element_type=_f32)
    at = jnp.dot(h, ac_ref[...], preferred_element_type=_f32)
    mt_ref[0] = jnp.concatenate([h[:, :128], at[:, :16]], axis=1)
    mt_ref[1] = jnp.concatenate([h[:, 128:], at[:, 16:32]], axis=1)
    atd_ref[0] = at[:, 32:48]
    atd_ref[1] = at[:, 48:64]


def _run_tc1(x, W1, acoef):
    B = 2000
    return pl.pallas_call(
        _tc1_body,
        grid=(N_NODES // B,),
        in_specs=[
            pl.BlockSpec((B, D_FEAT), lambda i: (i, 0)),
            pl.BlockSpec((D_FEAT, 256), lambda i: (0, 0)),
            pl.BlockSpec((256, 64), lambda i: (0, 0)),
        ],
        out_specs=[
            pl.BlockSpec((2, B, ACC1W), lambda i: (0, i, 0)),
            pl.BlockSpec((2, B, 16), lambda i: (0, i, 0)),
        ],
        out_shape=[
            jax.ShapeDtypeStruct((2, N_NODES, ACC1W), _f32),
            jax.ShapeDtypeStruct((2, N_NODES, 16), _f32),
        ],
    )(x, W1, acoef)


def _bcast_lane(vec, lane):
    idx = jnp.full((LANES,), lane, dtype=_i32)
    return vec.at[idx].get(mode="promise_in_bounds")


def _sc1_body(mt_hbm, atd_hbm, src3_hbm, dst3_hbm, zero_hbm, out_hbm,
              sidx0, sidx1, didx0, didx1, m0, m1, arow_d, acc,
              semi0, semi1, semg0, semg1, semd):
    c = lax.axis_index("c")
    s = lax.axis_index("s")
    r0 = s * RPT
    pltpu.sync_copy(zero_hbm.at[pl.ds(r0, RPT)], acc.at[pl.ds(r0, RPT)])
    plsc.subcore_barrier()

    mtab = mt_hbm.at[c]
    dtab = atd_hbm.at[c]
    semsI = (semi0, semi1)
    semsG = (semg0, semg1)
    sidxs = (sidx0, sidx1)
    didxs = (didx0, didx1)
    ms = (m0, m1)
    src_t = src3_hbm.at[s]
    dst_t = dst3_hbm.at[s]

    def issue_idx(i, q):
        pltpu.async_copy(src_t.at[i], sidxs[q], semsI[q])
        pltpu.async_copy(dst_t.at[i], didxs[q], semsI[q])

    def wait_idx(q):
        pltpu.make_async_copy(src_t.at[0], sidxs[q], semsI[q]).wait()
        pltpu.make_async_copy(dst_t.at[0], didxs[q], semsI[q]).wait()

    def issue_m(q):
        pltpu.async_copy(mtab.at[sidxs[q]], ms[q], semsG[q])

    def wait_m(q):
        pltpu.make_async_copy(mtab.at[sidxs[q]], ms[q], semsG[q]).wait()

    def issue_d(q):
        pltpu.async_copy(dtab.at[didxs[q]], arow_d, semd)

    def wait_d(q):
        pltpu.make_async_copy(dtab.at[didxs[q]], arow_d, semd).wait()

    issue_idx(0, 0)
    wait_idx(0)
    issue_m(0)
    issue_d(0)
    issue_idx(1, 1)

    def outer(t, carry):
        for q in range(2):
            i = 2 * t + q

            @pl.when(i + 1 < NCH1)
            def _():
                wait_idx(1 - q)
                issue_m(1 - q)

            wait_m(q)
            wait_d(q)
            m = ms[q]

            @plsc.parallel_loop(0, C1, step=1, unroll=4)
            def _edge(j):
                a = m[j, pl.ds(128, LANES)] + arow_d[j, :]
                a = jnp.maximum(a, 0.2 * a)
                ea = jnp.exp(a)
                m[j, pl.ds(128, LANES)] = ea
                for hd in range(4):
                    bc = _bcast_lane(ea, hd)
                    m[j, pl.ds(32 * hd, LANES)] = (
                        bc * m[j, pl.ds(32 * hd, LANES)])
                    m[j, pl.ds(32 * hd + 16, LANES)] = (
                        bc * m[j, pl.ds(32 * hd + 16, LANES)])

            @pl.when(i + 1 < NCH1)
            def _():
                issue_d(1 - q)

            pltpu.sync_copy(m, acc.at[didxs[q]], add=True)

            @pl.when(i + 2 < NCH1)
            def _():
                issue_idx(i + 2, q)
        return carry

    lax.fori_loop(0, NCH1 // 2, outer, 0)
    plsc.subcore_barrier()
    pltpu.sync_copy(acc.at[pl.ds(r0, RPT)], out_hbm.at[c].at[pl.ds(r0, RPT)])


def _run_sc1(mt, atd, src3, dst3, zeros1):
    mesh = plsc.VectorSubcoreMesh(core_axis_name="c", subcore_axis_name="s")
    kern = pl.kernel(
        _sc1_body,
        out_type=jax.ShapeDtypeStruct((NSC, NPAD, ACC1W), _f32),
        mesh=mesh,
        scratch_types=[
            pltpu.VMEM((C1,), _i32),
            pltpu.VMEM((C1,), _i32),
            pltpu.VMEM((C1,), _i32),
            pltpu.VMEM((C1,), _i32),
            pltpu.VMEM((C1, ACC1W), _f32),
            pltpu.VMEM((C1, ACC1W), _f32),
            pltpu.VMEM((C1, 16), _f32),
            pltpu.VMEM_SHARED((NPAD, ACC1W), _f32),
            pltpu.SemaphoreType.DMA,
            pltpu.SemaphoreType.DMA,
            pltpu.SemaphoreType.DMA,
            pltpu.SemaphoreType.DMA,
            pltpu.SemaphoreType.DMA,
        ],
        compiler_params=pltpu.CompilerParams(use_tc_tiling_on_sc=False),
    )
    return kern(mt, atd, src3, dst3, zeros1)


def _tc2a_body(acc_ref, b1_ref, h1_ref, sums_ref):
    i = pl.program_id(0)
    halves = []
    for cidx in range(2):
        blk = acc_ref[cidx]
        num = blk[:, :128]
        den = blk[:, 128:132]
        denb = jnp.concatenate(
            [jnp.broadcast_to(den[:, h:h + 1], (num.shape[0], 32)) for h in range(4)],
            axis=1)
        halves.append(num / (denb + 1e-16))
    h1 = jnp.concatenate(halves, axis=1) + b1_ref[...]
    h1_ref[...] = h1
    s1 = jnp.sum(h1, axis=0, keepdims=True)
    s2 = jnp.sum(h1 * h1, axis=0, keepdims=True)
    upd = jnp.concatenate([s1, s2, jnp.zeros((6, 256), _f32)], axis=0)

    @pl.when(i == 0)
    def _():
        sums_ref[...] = jnp.zeros((8, 256), _f32)

    sums_ref[...] += upd


def _run_tc2a(acc1, b1row):
    B = 2000
    return pl.pallas_call(
        _tc2a_body,
        grid=(N_NODES // B,),
        in_specs=[
            pl.BlockSpec((2, B, ACC1W), lambda i: (0, i, 0)),
            pl.BlockSpec((1, 256), lambda i: (0, 0)),
        ],
        out_specs=[
            pl.BlockSpec((B, 256), lambda i: (i, 0)),
            pl.BlockSpec((8, 256), lambda i: (0, 0)),
        ],
        out_shape=[
            jax.ShapeDtypeStruct((N_NODES, 256), _f32),
            jax.ShapeDtypeStruct((8, 256), _f32),
        ],
    )(acc1, b1row)


def _tc2b_body(h1_ref, sums_ref, g_ref, be_ref, w2_ref, d2_ref, p_ref, dt_ref):
    inv_n = 1.0 / N_NODES
    mu = sums_ref[0:1] * inv_n
    msq = sums_ref[1:2] * inv_n
    var = msq - mu * mu
    hn = (h1_ref[...] - mu) * lax.rsqrt(var + 1e-5) * g_ref[...] + be_ref[...]
    e = jnp.where(hn > 0, hn, jnp.exp(hn) - 1.0)
    p = jnp.dot(e, w2_ref[...], preferred_element_type=_f32)
    dt = jnp.dot(e, d2_ref[...], preferred_element_type=_f32)
    col = lax.broadcasted_iota(_i32, p.shape, 1)
    p_ref[...] = jnp.where(col == 41, 1.0, p)
    dt_ref[...] = dt


def _run_tc2b(h1, sums, grow, berow, W2e, d2coef):
    B = 2000
    return pl.pallas_call(
        _tc2b_body,
        grid=(N_NODES // B,),
        in_specs=[
            pl.BlockSpec((B, 256), lambda i: (i, 0)),
            pl.BlockSpec((8, 256), lambda i: (0, 0)),
            pl.BlockSpec((1, 256), lambda i: (0, 0)),
            pl.BlockSpec((1, 256), lambda i: (0, 0)),
            pl.BlockSpec((256, ACC2W), lambda i: (0, 0)),
            pl.BlockSpec((256, 16), lambda i: (0, 0)),
        ],
        out_specs=[
            pl.BlockSpec((B, ACC2W), lambda i: (i, 0)),
            pl.BlockSpec((B, 16), lambda i: (i, 0)),
        ],
        out_shape=[
            jax.ShapeDtypeStruct((N_NODES, ACC2W), _f32),
            jax.ShapeDtypeStruct((N_NODES, 16), _f32),
        ],
    )(h1, sums, grow, berow, W2e, d2coef)


def _sc2_body(ptab_hbm, dt_hbm, src3_hbm, dst3_hbm, zero_hbm, out_hbm,
              sidx0, sidx1, didx0, didx1, m0, m1, arow_d, acc,
              semi0, semi1, semg0, semg1, semd):
    c = lax.axis_index("c")
    s = lax.axis_index("s")
    r0 = s * RPT
    pltpu.sync_copy(zero_hbm.at[pl.ds(r0, RPT)], acc.at[pl.ds(r0, RPT)])
    plsc.subcore_barrier()

    w = c * NT + s
    semsI = (semi0, semi1)
    semsG = (semg0, semg1)
    sidxs = (sidx0, sidx1)
    didxs = (didx0, didx1)
    ms = (m0, m1)
    src_t = src3_hbm.at[w]
    dst_t = dst3_hbm.at[w]

    def issue_idx(i, q):
        pltpu.async_copy(src_t.at[i], sidxs[q], semsI[q])
        pltpu.async_copy(dst_t.at[i], didxs[q], semsI[q])

    def wait_idx(q):
        pltpu.make_async_copy(src_t.at[0], sidxs[q], semsI[q]).wait()
        pltpu.make_async_copy(dst_t.at[0], didxs[q], semsI[q]).wait()

    def issue_m(q):
        pltpu.async_copy(ptab_hbm.at[sidxs[q]], ms[q], semsG[q])

    def wait_m(q):
        pltpu.make_async_copy(ptab_hbm.at[sidxs[q]], ms[q], semsG[q]).wait()

    def issue_d(q):
        pltpu.async_copy(dt_hbm.at[didxs[q]], arow_d, semd)

    def wait_d(q):
        pltpu.make_async_copy(dt_hbm.at[didxs[q]], arow_d, semd).wait()

    issue_idx(0, 0)
    wait_idx(0)
    issue_m(0)
    issue_d(0)
    issue_idx(1, 1)

    def outer(t, carry):
        for q in range(2):
            i = 2 * t + q

            @pl.when(i + 1 < NCH2)
            def _():
                wait_idx(1 - q)
                issue_m(1 - q)

            wait_m(q)
            wait_d(q)
            m = ms[q]

            @plsc.parallel_loop(0, C2, step=1, unroll=4)
            def _edge(j):
                v2 = m[j, pl.ds(32, LANES)]
                a = _bcast_lane(v2, 8) + _bcast_lane(arow_d[j, :], 0)
                a = jnp.maximum(a, 0.2 * a)
                ea = jnp.exp(a)
                m[j, pl.ds(0, LANES)] = ea * m[j, pl.ds(0, LANES)]
                m[j, pl.ds(16, LANES)] = ea * m[j, pl.ds(16, LANES)]
                m[j, pl.ds(32, LANES)] = ea * v2
            @pl.when(i + 1 < NCH2)
            def _():
                issue_d(1 - q)

            pltpu.sync_copy(m, acc.at[didxs[q]], add=True)

            @pl.when(i + 2 < NCH2)
            def _():
                issue_idx(i + 2, q)
        return carry

    lax.fori_loop(0, NCH2 // 2, outer, 0)
    plsc.subcore_barrier()
    pltpu.sync_copy(acc.at[pl.ds(r0, RPT)], out_hbm.at[c].at[pl.ds(r0, RPT)])


def _run_sc2(ptab, dt, src3, dst3, zeros2):
    mesh = plsc.VectorSubcoreMesh(core_axis_name="c", subcore_axis_name="s")
    kern = pl.kernel(
        _sc2_body,
        out_type=jax.ShapeDtypeStruct((NSC, NPAD, ACC2W), _f32),
        mesh=mesh,
        scratch_types=[
            pltpu.VMEM((C2,), _i32),
            pltpu.VMEM((C2,), _i32),
            pltpu.VMEM((C2,), _i32),
            pltpu.VMEM((C2,), _i32),
            pltpu.VMEM((C2, ACC2W), _f32),
            pltpu.VMEM((C2, ACC2W), _f32),
            pltpu.VMEM((C2, 16), _f32),
            pltpu.VMEM_SHARED((NPAD, ACC2W), _f32),
            pltpu.SemaphoreType.DMA,
            pltpu.SemaphoreType.DMA,
            pltpu.SemaphoreType.DMA,
            pltpu.SemaphoreType.DMA,
            pltpu.SemaphoreType.DMA,
        ],
        compiler_params=pltpu.CompilerParams(use_tc_tiling_on_sc=False),
    )
    return kern(ptab, dt, src3, dst3, zeros2)


def _tc3_body(acc_ref, b2_ref, out_ref):
    ssum = acc_ref[0] + acc_ref[1]
    den = jnp.broadcast_to(ssum[:, 41:42], (ssum.shape[0], 40))
    out_ref[...] = ssum[:, :40] / (den + 1e-16) + b2_ref[...]


def _run_tc3(acc2, b2row):
    B = 2000
    return pl.pallas_call(
        _tc3_body,
        grid=(N_NODES // B,),
        in_specs=[
            pl.BlockSpec((2, B, ACC2W), lambda i: (0, i, 0)),
            pl.BlockSpec((1, 40), lambda i: (0, 0)),
        ],
        out_specs=pl.BlockSpec((B, 40), lambda i: (i, 0)),
        out_shape=jax.ShapeDtypeStruct((N_NODES, 40), _f32),
    )(acc2, b2row)


def kernel(x, edge_index, W1, a_src1, a_dst1, b1, gamma, beta, W2, a_src2, a_dst2, b2):
    acoef = jnp.zeros((HEADS1, CH1, 64), _f32)
    heads_idx = jnp.arange(HEADS1)
    j0 = (heads_idx // 4) * 16 + (heads_idx % 4)
    acoef = acoef.at[heads_idx, :, j0].set(a_src1)
    acoef = acoef.at[heads_idx, :, j0 + 32].set(a_dst1)
    acoef = acoef.reshape(HEADS1 * CH1, 64)

    W2e = jnp.zeros((256, ACC2W), _f32).at[:, :NUM_CLASSES].set(W2)
    W2e = W2e.at[:, NUM_CLASSES].set(W2 @ a_src2[0])
    d2coef = jnp.zeros((256, 16), _f32).at[:, 0].set(W2 @ a_dst2[0])

    npad_e = EP - N_EDGES
    srcp = jnp.concatenate([edge_index[0], jnp.zeros((npad_e,), _i32)])
    dstp = jnp.concatenate([edge_index[1], jnp.full((npad_e,), N_NODES, _i32)])
    src3_1 = srcp.reshape(NT, NCH1, C1)
    dst3_1 = dstp.reshape(NT, NCH1, C1)
    src3_2 = srcp.reshape(NSC * NT, NCH2, C2)
    dst3_2 = dstp.reshape(NSC * NT, NCH2, C2)

    zeros1 = jnp.zeros((NPAD, ACC1W), _f32)
    zeros2 = jnp.zeros((NPAD, ACC2W), _f32)
    b1row = b1.reshape(1, 256)
    grow = gamma.reshape(1, 256)
    berow = beta.reshape(1, 256)
    b2row = b2.reshape(1, NUM_CLASSES)

    mt, atd = _run_tc1(x, W1, acoef)
    mtp = jnp.concatenate([mt, jnp.zeros((2, NPAD - N_NODES, ACC1W), _f32)], axis=1)
    atdp = jnp.concatenate([atd, jnp.zeros((2, NPAD - N_NODES, 16), _f32)], axis=1)
    acc1 = _run_sc1(mtp, atdp, src3_1, dst3_1, zeros1)

    h1, sums = _run_tc2a(acc1[:, :N_NODES, :], b1row)
    ptab, dt = _run_tc2b(h1, sums, grow, berow, W2e, d2coef)
    ptabp = jnp.concatenate([ptab, jnp.zeros((NPAD - N_NODES, ACC2W), _f32)], axis=0)
    dtp = jnp.concatenate([dt, jnp.zeros((NPAD - N_NODES, 16), _f32)], axis=0)

    acc2 = _run_sc2(ptabp, dtp, src3_2, dst3_2, zeros2)
    out = _run_tc3(acc2[:, :N_NODES, :], b2row)
    return out


# speedup vs baseline: 1.7710x; 1.7710x over previous
import jax
import jax.numpy as jnp
from jax import lax
from jax.experimental import pallas as pl
from jax.experimental.pallas import tpu as pltpu, tpu_sc as plsc

N_NODES = 10000
N_EDGES = 320000
D_FEAT = 128
HEADS1 = 8
CH1 = 32
NUM_CLASSES = 40

NSC = 2
NT = 16
LANES = 16

C1 = 128
C2 = 128
EP = 327680
EPT1 = EP // NT
NCH1 = EPT1 // C1
EPT2 = EP // (NSC * NT)
NCH2 = EPT2 // C2

NPAD = 10112
RPT = NPAD // NT
ACC1W = 144
ACC2W = 64

_f32 = jnp.float32
_i32 = jnp.int32


def _tc1_body(x_ref, w1_ref, ac_ref, mt_ref, atd_ref):
    h = jnp.dot(x_ref[...], w1_ref[...], preferred_element_type=_f32)
    at = jnp.dot(h, ac_ref[...], preferred_element_type=_f32)
    mt_ref[0] = jnp.concatenate([h[:, :128], at[:, :16]], axis=1)
    mt_ref[1] = jnp.concatenate([h[:, 128:], at[:, 16:32]], axis=1)
    atd_ref[0] = at[:, 32:48]
    atd_ref[1] = at[:, 48:64]


def _run_tc1(x, W1, acoef):
    B = 2000
    return pl.pallas_call(
        _tc1_body,
        grid=(N_NODES // B,),
        in_specs=[
            pl.BlockSpec((B, D_FEAT), lambda i: (i, 0)),
            pl.BlockSpec((D_FEAT, 256), lambda i: (0, 0)),
            pl.BlockSpec((256, 64), lambda i: (0, 0)),
        ],
        out_specs=[
            pl.BlockSpec((2, B, ACC1W), lambda i: (0, i, 0)),
            pl.BlockSpec((2, B, 16), lambda i: (0, i, 0)),
        ],
        out_shape=[
            jax.ShapeDtypeStruct((2, NPAD, ACC1W), _f32),
            jax.ShapeDtypeStruct((2, NPAD, 16), _f32),
        ],
    )(x, W1, acoef)


def _bcast_lane(vec, lane):
    idx = jnp.full((LANES,), lane, dtype=_i32)
    return vec.at[idx].get(mode="promise_in_bounds")


def _sc1_body(mt_hbm, atd_hbm, src3_hbm, dst3_hbm, zero_hbm, out_hbm,
              sidx0, sidx1, didx0, didx1, m0, m1, arow_d, acc,
              semi0, semi1, semg0, semg1, semd):
    c = lax.axis_index("c")
    s = lax.axis_index("s")
    r0 = s * RPT
    pltpu.sync_copy(zero_hbm.at[pl.ds(r0, RPT)], acc.at[pl.ds(r0, RPT)])
    plsc.subcore_barrier()

    mtab = mt_hbm.at[c]
    dtab = atd_hbm.at[c]
    semsI = (semi0, semi1)
    semsG = (semg0, semg1)
    sidxs = (sidx0, sidx1)
    didxs = (didx0, didx1)
    ms = (m0, m1)
    src_t = src3_hbm.at[s]
    dst_t = dst3_hbm.at[s]

    def issue_idx(i, q):
        pltpu.async_copy(src_t.at[i], sidxs[q], semsI[q])
        pltpu.async_copy(dst_t.at[i], didxs[q], semsI[q])

    def wait_idx(q):
        pltpu.make_async_copy(src_t.at[0], sidxs[q], semsI[q]).wait()
        pltpu.make_async_copy(dst_t.at[0], didxs[q], semsI[q]).wait()

    def issue_m(q):
        pltpu.async_copy(mtab.at[sidxs[q]], ms[q], semsG[q])

    def wait_m(q):
        pltpu.make_async_copy(mtab.at[sidxs[q]], ms[q], semsG[q]).wait()

    def issue_d(q):
        pltpu.async_copy(dtab.at[didxs[q]], arow_d, semd)

    def wait_d(q):
        pltpu.make_async_copy(dtab.at[didxs[q]], arow_d, semd).wait()

    issue_idx(0, 0)
    wait_idx(0)
    issue_m(0)
    issue_d(0)
    issue_idx(1, 1)

    def outer(t, carry):
        for q in range(2):
            i = 2 * t + q

            @pl.when(i + 1 < NCH1)
            def _():
                wait_idx(1 - q)
                issue_m(1 - q)

            wait_m(q)
            wait_d(q)
            m = ms[q]

            @plsc.parallel_loop(0, C1, step=1, unroll=4)
            def _edge(j):
                a = m[j, pl.ds(128, LANES)] + arow_d[j, :]
                a = jnp.maximum(a, 0.2 * a)
                ea = jnp.exp(a)
                m[j, pl.ds(128, LANES)] = ea
                for hd in range(4):
                    bc = _bcast_lane(ea, hd)
                    m[j, pl.ds(32 * hd, LANES)] = (
                        bc * m[j, pl.ds(32 * hd, LANES)])
                    m[j, pl.ds(32 * hd + 16, LANES)] = (
                        bc * m[j, pl.ds(32 * hd + 16, LANES)])

            @pl.when(i + 1 < NCH1)
            def _():
                issue_d(1 - q)

            pltpu.sync_copy(m, acc.at[didxs[q]], add=True)

            @pl.when(i + 2 < NCH1)
            def _():
                issue_idx(i + 2, q)
        return carry

    lax.fori_loop(0, NCH1 // 2, outer, 0)
    plsc.subcore_barrier()
    pltpu.sync_copy(acc.at[pl.ds(r0, RPT)], out_hbm.at[c].at[pl.ds(r0, RPT)])


def _run_sc1(mt, atd, src3, dst3, zeros1):
    mesh = plsc.VectorSubcoreMesh(core_axis_name="c", subcore_axis_name="s")
    kern = pl.kernel(
        _sc1_body,
        out_type=jax.ShapeDtypeStruct((NSC, NPAD, ACC1W), _f32),
        mesh=mesh,
        scratch_types=[
            pltpu.VMEM((C1,), _i32),
            pltpu.VMEM((C1,), _i32),
            pltpu.VMEM((C1,), _i32),
            pltpu.VMEM((C1,), _i32),
            pltpu.VMEM((C1, ACC1W), _f32),
            pltpu.VMEM((C1, ACC1W), _f32),
            pltpu.VMEM((C1, 16), _f32),
            pltpu.VMEM_SHARED((NPAD, ACC1W), _f32),
            pltpu.SemaphoreType.DMA,
            pltpu.SemaphoreType.DMA,
            pltpu.SemaphoreType.DMA,
            pltpu.SemaphoreType.DMA,
            pltpu.SemaphoreType.DMA,
        ],
        compiler_params=pltpu.CompilerParams(use_tc_tiling_on_sc=False),
    )
    return kern(mt, atd, src3, dst3, zeros1)


def _tc2a_body(acc_ref, b1_ref, h1_ref, sums_ref):
    i = pl.program_id(0)
    halves = []
    for cidx in range(2):
        blk = acc_ref[cidx]
        num = blk[:, :128]
        den = blk[:, 128:132]
        denb = jnp.concatenate(
            [jnp.broadcast_to(den[:, h:h + 1], (num.shape[0], 32)) for h in range(4)],
            axis=1)
        halves.append(num / (denb + 1e-16))
    h1 = jnp.concatenate(halves, axis=1) + b1_ref[...]
    h1_ref[...] = h1
    s1 = jnp.sum(h1, axis=0, keepdims=True)
    s2 = jnp.sum(h1 * h1, axis=0, keepdims=True)
    upd = jnp.concatenate([s1, s2, jnp.zeros((6, 256), _f32)], axis=0)

    @pl.when(i == 0)
    def _():
        sums_ref[...] = jnp.zeros((8, 256), _f32)

    sums_ref[...] += upd


def _run_tc2a(acc1, b1row):
    B = 2000
    return pl.pallas_call(
        _tc2a_body,
        grid=(N_NODES // B,),
        in_specs=[
            pl.BlockSpec((2, B, ACC1W), lambda i: (0, i, 0)),
            pl.BlockSpec((1, 256), lambda i: (0, 0)),
        ],
        out_specs=[
            pl.BlockSpec((B, 256), lambda i: (i, 0)),
            pl.BlockSpec((8, 256), lambda i: (0, 0)),
        ],
        out_shape=[
            jax.ShapeDtypeStruct((N_NODES, 256), _f32),
            jax.ShapeDtypeStruct((8, 256), _f32),
        ],
    )(acc1, b1row)


def _tc2b_body(h1_ref, sums_ref, g_ref, be_ref, w2_ref, d2_ref, p_ref, dt_ref):
    inv_n = 1.0 / N_NODES
    mu = sums_ref[0:1] * inv_n
    msq = sums_ref[1:2] * inv_n
    var = msq - mu * mu
    hn = (h1_ref[...] - mu) * lax.rsqrt(var + 1e-5) * g_ref[...] + be_ref[...]
    e = jnp.where(hn > 0, hn, jnp.exp(hn) - 1.0)
    p = jnp.dot(e, w2_ref[...], preferred_element_type=_f32)
    dt = jnp.dot(e, d2_ref[...], preferred_element_type=_f32)
    col = lax.broadcasted_iota(_i32, p.shape, 1)
    p_ref[...] = jnp.where(col == 41, 1.0, p)
    dt_ref[...] = dt


def _run_tc2b(h1, sums, grow, berow, W2e, d2coef):
    B = 2000
    return pl.pallas_call(
        _tc2b_body,
        grid=(N_NODES // B,),
        in_specs=[
            pl.BlockSpec((B, 256), lambda i: (i, 0)),
            pl.BlockSpec((8, 256), lambda i: (0, 0)),
            pl.BlockSpec((1, 256), lambda i: (0, 0)),
            pl.BlockSpec((1, 256), lambda i: (0, 0)),
            pl.BlockSpec((256, ACC2W), lambda i: (0, 0)),
            pl.BlockSpec((256, 16), lambda i: (0, 0)),
        ],
        out_specs=[
            pl.BlockSpec((B, ACC2W), lambda i: (i, 0)),
            pl.BlockSpec((B, 16), lambda i: (i, 0)),
        ],
        out_shape=[
            jax.ShapeDtypeStruct((NPAD, ACC2W), _f32),
            jax.ShapeDtypeStruct((NPAD, 16), _f32),
        ],
    )(h1, sums, grow, berow, W2e, d2coef)


def _sc2_body(ptab_hbm, dt_hbm, src3_hbm, dst3_hbm, zero_hbm, out_hbm,
              sidx0, sidx1, didx0, didx1, m0, m1, arow_d, acc,
              semi0, semi1, semg0, semg1, semd):
    c = lax.axis_index("c")
    s = lax.axis_index("s")
    r0 = s * RPT
    pltpu.sync_copy(zero_hbm.at[pl.ds(r0, RPT)], acc.at[pl.ds(r0, RPT)])
    plsc.subcore_barrier()

    w = c * NT + s
    semsI = (semi0, semi1)
    semsG = (semg0, semg1)
    sidxs = (sidx0, sidx1)
    didxs = (didx0, didx1)
    ms = (m0, m1)
    src_t = src3_hbm.at[w]
    dst_t = dst3_hbm.at[w]

    def issue_idx(i, q):
        pltpu.async_copy(src_t.at[i], sidxs[q], semsI[q])
        pltpu.async_copy(dst_t.at[i], didxs[q], semsI[q])

    def wait_idx(q):
        pltpu.make_async_copy(src_t.at[0], sidxs[q], semsI[q]).wait()
        pltpu.make_async_copy(dst_t.at[0], didxs[q], semsI[q]).wait()

    def issue_m(q):
        pltpu.async_copy(ptab_hbm.at[sidxs[q]], ms[q], semsG[q])

    def wait_m(q):
        pltpu.make_async_copy(ptab_hbm.at[sidxs[q]], ms[q], semsG[q]).wait()

    def issue_d(q):
        pltpu.async_copy(dt_hbm.at[didxs[q]], arow_d, semd)

    def wait_d(q):
        pltpu.make_async_copy(dt_hbm.at[didxs[q]], arow_d, semd).wait()

    issue_idx(0, 0)
    wait_idx(0)
    issue_m(0)
    issue_d(0)
    issue_idx(1, 1)

    def outer(t, carry):
        for q in range(2):
            i = 2 * t + q

            @pl.when(i + 1 < NCH2)
            def _():
                wait_idx(1 - q)
                issue_m(1 - q)

            wait_m(q)
            wait_d(q)
            m = ms[q]

            @plsc.parallel_loop(0, C2, step=1, unroll=4)
            def _edge(j):
                v2 = m[j, pl.ds(32, LANES)]
                a = _bcast_lane(v2, 8) + _bcast_lane(arow_d[j, :], 0)
                a = jnp.maximum(a, 0.2 * a)
                ea = jnp.exp(a)
                m[j, pl.ds(0, LANES)] = ea * m[j, pl.ds(0, LANES)]
                m[j, pl.ds(16, LANES)] = ea * m[j, pl.ds(16, LANES)]
                m[j, pl.ds(32, LANES)] = ea * v2
            @pl.when(i + 1 < NCH2)
            def _():
                issue_d(1 - q)

            pltpu.sync_copy(m, acc.at[didxs[q]], add=True)

            @pl.when(i + 2 < NCH2)
            def _():
                issue_idx(i + 2, q)
        return carry

    lax.fori_loop(0, NCH2 // 2, outer, 0)
    plsc.subcore_barrier()
    pltpu.sync_copy(acc.at[pl.ds(r0, RPT)], out_hbm.at[c].at[pl.ds(r0, RPT)])


def _run_sc2(ptab, dt, src3, dst3, zeros2):
    mesh = plsc.VectorSubcoreMesh(core_axis_name="c", subcore_axis_name="s")
    kern = pl.kernel(
        _sc2_body,
        out_type=jax.ShapeDtypeStruct((NSC, NPAD, ACC2W), _f32),
        mesh=mesh,
        scratch_types=[
            pltpu.VMEM((C2,), _i32),
            pltpu.VMEM((C2,), _i32),
            pltpu.VMEM((C2,), _i32),
            pltpu.VMEM((C2,), _i32),
            pltpu.VMEM((C2, ACC2W), _f32),
            pltpu.VMEM((C2, ACC2W), _f32),
            pltpu.VMEM((C2, 16), _f32),
            pltpu.VMEM_SHARED((NPAD, ACC2W), _f32),
            pltpu.SemaphoreType.DMA,
            pltpu.SemaphoreType.DMA,
            pltpu.SemaphoreType.DMA,
            pltpu.SemaphoreType.DMA,
            pltpu.SemaphoreType.DMA,
        ],
        compiler_params=pltpu.CompilerParams(use_tc_tiling_on_sc=False),
    )
    return kern(ptab, dt, src3, dst3, zeros2)


def _tc3_body(acc_ref, b2_ref, out_ref):
    ssum = acc_ref[0] + acc_ref[1]
    den = jnp.broadcast_to(ssum[:, 41:42], (ssum.shape[0], 40))
    out_ref[...] = ssum[:, :40] / (den + 1e-16) + b2_ref[...]


def _run_tc3(acc2, b2row):
    B = 2000
    return pl.pallas_call(
        _tc3_body,
        grid=(N_NODES // B,),
        in_specs=[
            pl.BlockSpec((2, B, ACC2W), lambda i: (0, i, 0)),
            pl.BlockSpec((1, 40), lambda i: (0, 0)),
        ],
        out_specs=pl.BlockSpec((B, 40), lambda i: (i, 0)),
        out_shape=jax.ShapeDtypeStruct((N_NODES, 40), _f32),
    )(acc2, b2row)


def kernel(x, edge_index, W1, a_src1, a_dst1, b1, gamma, beta, W2, a_src2, a_dst2, b2):
    acoef = jnp.zeros((HEADS1, CH1, 64), _f32)
    heads_idx = jnp.arange(HEADS1)
    j0 = (heads_idx // 4) * 16 + (heads_idx % 4)
    acoef = acoef.at[heads_idx, :, j0].set(a_src1)
    acoef = acoef.at[heads_idx, :, j0 + 32].set(a_dst1)
    acoef = acoef.reshape(HEADS1 * CH1, 64)

    W2e = jnp.zeros((256, ACC2W), _f32).at[:, :NUM_CLASSES].set(W2)
    W2e = W2e.at[:, NUM_CLASSES].set(W2 @ a_src2[0])
    d2coef = jnp.zeros((256, 16), _f32).at[:, 0].set(W2 @ a_dst2[0])

    npad_e = EP - N_EDGES
    k = jnp.arange(npad_e, dtype=_i32)
    srcp = jnp.concatenate([edge_index[0], k % N_NODES])
    dstp = jnp.concatenate([edge_index[1], N_NODES + k % (NPAD - N_NODES)])
    src3_1 = srcp.reshape(NT, NCH1, C1)
    dst3_1 = dstp.reshape(NT, NCH1, C1)
    src3_2 = srcp.reshape(NSC * NT, NCH2, C2)
    dst3_2 = dstp.reshape(NSC * NT, NCH2, C2)

    zeros1 = jnp.zeros((NPAD, ACC1W), _f32)
    zeros2 = jnp.zeros((NPAD, ACC2W), _f32)
    b1row = b1.reshape(1, 256)
    grow = gamma.reshape(1, 256)
    berow = beta.reshape(1, 256)
    b2row = b2.reshape(1, NUM_CLASSES)

    mt, atd = _run_tc1(x, W1, acoef)
    acc1 = _run_sc1(mt, atd, src3_1, dst3_1, zeros1)

    h1, sums = _run_tc2a(acc1, b1row)
    ptab, dt = _run_tc2b(h1, sums, grow, berow, W2e, d2coef)

    acc2 = _run_sc2(ptab, dt, src3_2, dst3_2, zeros2)
    out = _run_tc3(acc2, b2row)
    return out
